# Initial kernel scaffold; baseline (speedup 1.0000x reference)
import jax
import jax.numpy as jnp
from jax.experimental import pallas as pl


def kernel(node_features, edge_index, symbolic_features, params):
    raise NotImplementedError("write your pallas kernel here")


# speedup vs baseline: 3.5961x; 3.5961x over previous
import functools
import math

import jax
import jax.numpy as jnp
from jax import lax
from jax.experimental import pallas as pl
from jax.experimental.pallas import tpu as pltpu
from jax.experimental.pallas import tpu_sc as plsc

N = 10000
NP = 10240
D = 128
NE = 320000
LAYERS = 3

NSC = 2
NTILE = 16
NW = NSC * NTILE
CHUNK = 128
CPT = 79
NE_PAD = NW * CPT * CHUNK
PAD = NE_PAD - NE
RPT = NP // NTILE
RCH = RPT // CHUNK

BLK = 1000

_SC_MESH = dict(core_axis_name="c", subcore_axis_name="s")


def _sc_agg(x, src_p, dst_p):
    @functools.partial(
        pl.kernel,
        out_type=jax.ShapeDtypeStruct((NSC, NP, D), jnp.float32),
        mesh=plsc.VectorSubcoreMesh(**_SC_MESH),
        scratch_types=[
            pltpu.VMEM((CHUNK,), jnp.int32),
            pltpu.VMEM((1, CHUNK), jnp.int32),
            pltpu.VMEM((CHUNK, D), jnp.float32),
            pltpu.VMEM_SHARED((NP, D), jnp.float32),
        ],
    )
    def k(x_hbm, src_hbm, dst_hbm, out_hbm, src_v, dst_v, rows_v, acc_sh):
        c = lax.axis_index("c")
        s = lax.axis_index("s")
        wid = c * NTILE + s

        zv = jnp.zeros((16,), jnp.float32)

        @pl.loop(0, CHUNK)
        def _zr(r):
            @pl.loop(0, D // 16)
            def _zc(l):
                rows_v[r, pl.ds(l * 16, 16)] = zv

        @pl.loop(0, RCH)
        def _z(i):
            pltpu.sync_copy(rows_v, acc_sh.at[pl.ds(s * RPT + i * CHUNK, CHUNK)])

        plsc.subcore_barrier()

        base = wid * CPT * CHUNK

        @pl.loop(0, CPT)
        def _edge(j):
            off = base + j * CHUNK
            pltpu.sync_copy(src_hbm.at[pl.ds(off, CHUNK)], src_v)
            pltpu.sync_copy(dst_hbm.at[pl.ds(off, CHUNK)], dst_v.at[0])
            pltpu.sync_copy(x_hbm.at[src_v], rows_v)
            pltpu.sync_copy(rows_v, acc_sh.at[dst_v.at[0]], add=True)

        plsc.subcore_barrier()

        @pl.loop(0, RCH)
        def _out(i):
            sl = pl.ds(s * RPT + i * CHUNK, CHUNK)
            pltpu.sync_copy(acc_sh.at[sl], rows_v)
            pltpu.sync_copy(rows_v, out_hbm.at[c, sl])

    return k(x, src_p, dst_p)


def _sc_deg(dst_p):
    @functools.partial(
        pl.kernel,
        out_type=jax.ShapeDtypeStruct((NSC, NP, D), jnp.float32),
        mesh=plsc.VectorSubcoreMesh(**_SC_MESH),
        scratch_types=[
            pltpu.VMEM((1, CHUNK), jnp.int32),
            pltpu.VMEM((CHUNK, D), jnp.float32),
            pltpu.VMEM_SHARED((NP, D), jnp.float32),
        ],
    )
    def k(dst_hbm, out_hbm, dst_v, ones_v, acc_sh):
        c = lax.axis_index("c")
        s = lax.axis_index("s")
        wid = c * NTILE + s

        zv = jnp.zeros((16,), jnp.float32)

        @pl.loop(0, CHUNK)
        def _zr(r):
            @pl.loop(0, D // 16)
            def _zc(l):
                ones_v[r, pl.ds(l * 16, 16)] = zv

        @pl.loop(0, RCH)
        def _z(i):
            pltpu.sync_copy(ones_v, acc_sh.at[pl.ds(s * RPT + i * CHUNK, CHUNK)])

        ov = jnp.ones((16,), jnp.float32)

        @pl.loop(0, CHUNK)
        def _or(r):
            @pl.loop(0, D // 16)
            def _oc(l):
                ones_v[r, pl.ds(l * 16, 16)] = ov

        plsc.subcore_barrier()

        base = wid * CPT * CHUNK

        @pl.loop(0, CPT)
        def _edge(j):
            off = base + j * CHUNK
            pltpu.sync_copy(dst_hbm.at[pl.ds(off, CHUNK)], dst_v.at[0])
            pltpu.sync_copy(ones_v, acc_sh.at[dst_v.at[0]], add=True)

        plsc.subcore_barrier()

        @pl.loop(0, RCH)
        def _out(i):
            sl = pl.ds(s * RPT + i * CHUNK, CHUNK)
            pltpu.sync_copy(acc_sh.at[sl], ones_v)
            pltpu.sync_copy(ones_v, out_hbm.at[c, sl])

    return k(dst_p)


def _tc_inproj(nf, w, b):
    def body(nf_ref, w_ref, b_ref, o_ref):
        o_ref[...] = jnp.maximum(
            jnp.dot(nf_ref[...], w_ref[...], preferred_element_type=jnp.float32)
            + b_ref[...], 0.0)

    return pl.pallas_call(
        body,
        grid=(N // BLK,),
        in_specs=[
            pl.BlockSpec((BLK, D), lambda i: (i, 0)),
            pl.BlockSpec((D, D), lambda i: (0, 0)),
            pl.BlockSpec((1, D), lambda i: (0, 0)),
        ],
        out_specs=pl.BlockSpec((BLK, D), lambda i: (i, 0)),
        out_shape=jax.ShapeDtypeStruct((N, D), jnp.float32),
    )(nf, w, b.reshape(1, D))


def _tc_layer(x, aggp, degp, wl, bl, wr, g, b):
    def body(x_ref, a_ref, d_ref, wl_ref, bl_ref, wr_ref, g_ref, b_ref, o_ref):
        agg = a_ref[0] + a_ref[1]
        deg = jnp.maximum(d_ref[0, :, 0:1] + d_ref[1, :, 0:1], 1.0)
        xv = x_ref[...]
        h = (jnp.dot(agg / deg, wl_ref[...], preferred_element_type=jnp.float32)
             + jnp.dot(xv, wr_ref[...], preferred_element_type=jnp.float32)
             + bl_ref[...])
        m = jnp.mean(h, axis=-1, keepdims=True)
        v = jnp.mean((h - m) ** 2, axis=-1, keepdims=True)
        h = (h - m) * lax.rsqrt(v + 1e-5) * g_ref[...] + b_ref[...]
        o_ref[...] = jnp.maximum(h, 0.0) + xv

    return pl.pallas_call(
        body,
        grid=(N // BLK,),
        in_specs=[
            pl.BlockSpec((BLK, D), lambda i: (i, 0)),
            pl.BlockSpec((NSC, BLK, D), lambda i: (0, i, 0)),
            pl.BlockSpec((NSC, BLK, D), lambda i: (0, i, 0)),
            pl.BlockSpec((D, D), lambda i: (0, 0)),
            pl.BlockSpec((1, D), lambda i: (0, 0)),
            pl.BlockSpec((D, D), lambda i: (0, 0)),
            pl.BlockSpec((1, D), lambda i: (0, 0)),
            pl.BlockSpec((1, D), lambda i: (0, 0)),
        ],
        out_specs=pl.BlockSpec((BLK, D), lambda i: (i, 0)),
        out_shape=jax.ShapeDtypeStruct((N, D), jnp.float32),
    )(x, aggp, degp, wl, bl.reshape(1, D), wr, g.reshape(1, D), b.reshape(1, D))


def _tc_final(x, sym_in, qry, wqT, bq, wkT, bk, wvT, bv, owT, ob,
              symw, symb, sfw, sfb, slg, slb, fw, fb, flg, flb,
              hw1, hb1, hw2, hb2):
    inv_sqrt_dh = 1.0 / math.sqrt(32.0)

    def _ln(h, gv, bv_):
        m = jnp.mean(h, axis=-1, keepdims=True)
        v = jnp.mean((h - m) ** 2, axis=-1, keepdims=True)
        return (h - m) * lax.rsqrt(v + 1e-5) * gv + bv_

    def body(x_ref, sym_ref, qry_ref, wqT_ref, bq_ref, wkT_ref, bk_ref,
             wvT_ref, bv_ref, owT_ref, ob_ref, symw_ref, symb_ref, sfw_ref,
             sfb_ref, slg_ref, slb_ref, fw_ref, fb_ref, flg_ref, flb_ref,
             hw1_ref, hb1_ref, hw2_ref, hb2_ref, o_ref):
        xv = x_ref[...]
        q = jnp.dot(qry_ref[...], wqT_ref[...],
                    preferred_element_type=jnp.float32) + bq_ref[...]
        K = jnp.dot(xv, wkT_ref[...], preferred_element_type=jnp.float32) + bk_ref[...]
        V = jnp.dot(xv, wvT_ref[...], preferred_element_type=jnp.float32) + bv_ref[...]
        att = []
        for h in range(4):
            sl = slice(h * 32, (h + 1) * 32)
            sc = jnp.sum(K[:, sl] * q[:, sl], axis=-1, keepdims=True) * inv_sqrt_dh
            sc = sc - jnp.max(sc)
            e = jnp.exp(sc)
            w = e / jnp.sum(e)
            att.append(jnp.sum(V[:, sl] * w, axis=0, keepdims=True))
        att = jnp.concatenate(att, axis=-1)
        graph_emb = jnp.dot(att, owT_ref[...],
                            preferred_element_type=jnp.float32) + ob_ref[...]

        symh = jnp.maximum(
            jnp.dot(sym_ref[...], symw_ref[...],
                    preferred_element_type=jnp.float32) + symb_ref[...], 0.0)
        symh = _ln(jnp.maximum(
            jnp.dot(symh, sfw_ref[...], preferred_element_type=jnp.float32)
            + sfb_ref[...], 0.0), slg_ref[...], slb_ref[...])

        comb = jnp.concatenate([graph_emb, symh], axis=-1)
        fused = _ln(jnp.maximum(
            jnp.dot(comb, fw_ref[...], preferred_element_type=jnp.float32)
            + fb_ref[...], 0.0), flg_ref[...], flb_ref[...])

        t = jnp.maximum(
            jnp.dot(fused, hw1_ref[...], preferred_element_type=jnp.float32)
            + hb1_ref[...], 0.0)
        logits = jnp.dot(t, hw2_ref[...],
                         preferred_element_type=jnp.float32) + hb2_ref[...]
        o_ref[...] = 1.0 / (1.0 + jnp.exp(-logits))

    full = lambda a: pl.BlockSpec(a.shape, lambda: tuple(0 for _ in a.shape))
    args = (x, sym_in, qry, wqT, bq, wkT, bk, wvT, bv, owT, ob,
            symw, symb, sfw, sfb, slg, slb, fw, fb, flg, flb,
            hw1, hb1, hw2, hb2)
    return pl.pallas_call(
        body,
        in_specs=[full(a) for a in args],
        out_specs=pl.BlockSpec((1, 128), lambda: (0, 0)),
        out_shape=jax.ShapeDtypeStruct((1, 128), jnp.float32),
    )(*args)


def kernel(node_features, edge_index, symbolic_features, params):
    p = params
    src = edge_index[0].astype(jnp.int32)
    dst = edge_index[1].astype(jnp.int32)
    src_p = jnp.concatenate([src, jnp.zeros((PAD,), jnp.int32)])
    dst_p = jnp.concatenate([dst, jnp.full((PAD,), N, jnp.int32)])

    degp = _sc_deg(dst_p)
    x = _tc_inproj(node_features, p["input_proj_w"], p["input_proj_b"])
    for l in range(LAYERS):
        aggp = _sc_agg(x, src_p, dst_p)
        x = _tc_layer(x, aggp, degp,
                      p[f"sage{l}_wl"], p[f"sage{l}_bl"], p[f"sage{l}_wr"],
                      p[f"ln{l}_g"], p[f"ln{l}_b"])

    in_w, in_b = p["attn_in_w"], p["attn_in_b"]
    wqT, wkT, wvT = in_w[:D].T, in_w[D:2 * D].T, in_w[2 * D:].T
    bq, bk, bv = in_b[:D].reshape(1, D), in_b[D:2 * D].reshape(1, D), in_b[2 * D:].reshape(1, D)
    owT = p["attn_out_w"].T
    ob = p["attn_out_b"].reshape(1, D)

    symw = jnp.zeros((32, D), jnp.float32)
    symb = jnp.zeros((1, D), jnp.float32)
    for s in range(4):
        symw = symw.at[s * 8:(s + 1) * 8, s * 32:(s + 1) * 32].set(p[f"sym{s}_w"])
        symb = symb.at[0, s * 32:(s + 1) * 32].set(p[f"sym{s}_b"])

    heads = ["inst", "grnd", "risk"]
    hw1 = jnp.concatenate([p[f"{h}_w1"] for h in heads], axis=1)
    hb1 = jnp.concatenate([p[f"{h}_b1"] for h in heads]).reshape(1, 192)
    hw2 = jnp.zeros((192, 128), jnp.float32)
    hb2 = jnp.zeros((1, 128), jnp.float32)
    for i, h in enumerate(heads):
        hw2 = hw2.at[i * 64:(i + 1) * 64, i].set(p[f"{h}_w2"][:, 0])
        hb2 = hb2.at[0, i].set(p[f"{h}_b2"][0])

    out = _tc_final(x, symbolic_features, p["pool_query"].reshape(1, D),
                    wqT, bq, wkT, bk, wvT, bv, owT, ob,
                    symw, symb, p["symf_w"], p["symf_b"].reshape(1, D),
                    p["symln_g"].reshape(1, D), p["symln_b"].reshape(1, D),
                    p["fuse_w"], p["fuse_b"].reshape(1, D),
                    p["fln_g"].reshape(1, D), p["fln_b"].reshape(1, D),
                    hw1, hb1, hw2, hb2)
    return (out[:, 0:1], out[:, 1:2], out[:, 2:3])
